# Initial kernel scaffold; baseline (speedup 1.0000x reference)
import jax
import jax.numpy as jnp
from jax.experimental import pallas as pl


def kernel(x, t):
    raise NotImplementedError("write your pallas kernel here")


# speedup vs baseline: 6.7329x; 6.7329x over previous
import numpy as np
import jax
import jax.numpy as jnp
from jax import lax
from jax.experimental import pallas as pl
from jax.experimental.pallas import tpu as pltpu

_B, _N, _D = 8, 4096, 1024
_L1, _L2 = 128, 64
_L = _L1 * _L2
_N1 = _N // _L2
_DBLK = 128


def _consts():
    n1 = np.arange(_N1)[:, None]
    k1 = np.arange(_L1)[None, :]
    F1 = np.exp(-2j * np.pi * n1 * k1 / _L1)
    n2 = np.arange(_L2)[:, None]
    Tw1 = np.exp(-2j * np.pi * n2 * k1 / _L)
    k2 = np.arange(_L2)[:, None]
    n2r = np.arange(_L2)[None, :]
    F2t = np.exp(-2j * np.pi * k2 * n2r / _L2)
    m2 = np.arange(_L2)[:, None]
    k2r = np.arange(_L2)[None, :]
    G2t = np.exp(+2j * np.pi * m2 * k2r / _L2)
    Tw2 = np.exp(+2j * np.pi * m2 * k1 / _L)
    k1c = np.arange(_L1)[:, None]
    m1 = np.arange(_N1)[None, :]
    G1 = np.exp(+2j * np.pi * k1c * m1 / _L1) / _L
    mats = []
    for m in (F1, Tw1, F2t, G2t, Tw2, G1):
        mats.append(np.ascontiguousarray(m.real.astype(np.float32)))
        mats.append(np.ascontiguousarray(m.imag.astype(np.float32)))
    return mats


_CONSTS = _consts()


def _dot0(a2, b3):
    return lax.dot_general(a2, b3, (((1,), (0,)), ((), ())),
                           preferred_element_type=jnp.float32)


def _dot_lhs0(a3, b2):
    return lax.dot_general(a3, b2, (((0,), (0,)), ((), ())),
                           preferred_element_type=jnp.float32)


def _dot_last(a3, b2):
    return lax.dot_general(a3, b2, (((2,), (0,)), ((), ())),
                           preferred_element_type=jnp.float32)


def _fwd(v, f1r, f1i, tw1r, tw1i, f2tr, f2ti):
    ar = _dot_lhs0(v, f1r)
    ai = _dot_lhs0(v, f1i)
    br = ar * tw1r - ai * tw1i
    bi = ar * tw1i + ai * tw1r
    pr = _dot0(f2tr, br) - _dot0(f2ti, bi)
    pi = _dot0(f2tr, bi) + _dot0(f2ti, br)
    return pr, pi


def _body(x_ref, t_ref,
          f1r_ref, f1i_ref, tw1r_ref, tw1i_ref, f2tr_ref, f2ti_ref,
          g2tr_ref, g2ti_ref, tw2r_ref, tw2i_ref, g1r_ref, g1i_ref,
          out_ref, ttr_ref, tti_ref):
    b = pl.program_id(1)
    tw1r = tw1r_ref[...].reshape(_L2, 1, _L1)
    tw1i = tw1i_ref[...].reshape(_L2, 1, _L1)

    @pl.when(b == 0)
    def _():
        pr, pi = _fwd(t_ref[...], f1r_ref[...], f1i_ref[...], tw1r, tw1i,
                      f2tr_ref[...], f2ti_ref[...])
        ttr_ref[...] = pr
        tti_ref[...] = pi

    xr, xi = _fwd(x_ref[0], f1r_ref[...], f1i_ref[...], tw1r, tw1i,
                  f2tr_ref[...], f2ti_ref[...])
    ttr = ttr_ref[...]
    tti = tti_ref[...]
    pwr = xr * ttr - xi * tti
    pwi = xr * tti + xi * ttr

    g2tr = g2tr_ref[...]
    g2ti = g2ti_ref[...]
    qr = _dot0(g2tr, pwr) - _dot0(g2ti, pwi)
    qi = _dot0(g2tr, pwi) + _dot0(g2ti, pwr)
    tw2r = tw2r_ref[...].reshape(_L2, 1, _L1)
    tw2i = tw2i_ref[...].reshape(_L2, 1, _L1)
    rr = qr * tw2r - qi * tw2i
    ri = qr * tw2i + qi * tw2r
    yr = _dot_last(rr, g1r_ref[...]) - _dot_last(ri, g1i_ref[...])
    out_ref[0] = yr


def kernel(x, t):
    b, n, d = x.shape
    xv = x.reshape(b, _N1, _L2, d)
    tv = t.reshape(_N1, _L2, d)
    consts = [jnp.asarray(m) for m in _CONSTS]
    nd = d // _DBLK

    def cmap(shape):
        return pl.BlockSpec(shape, lambda di, bi: tuple(0 for _ in shape))

    in_specs = [
        pl.BlockSpec((1, _N1, _L2, _DBLK), lambda di, bi: (bi, 0, 0, di)),
        pl.BlockSpec((_N1, _L2, _DBLK), lambda di, bi: (0, 0, di)),
        cmap((_N1, _L1)), cmap((_N1, _L1)),
        cmap((_L2, _L1)), cmap((_L2, _L1)),
        cmap((_L2, _L2)), cmap((_L2, _L2)),
        cmap((_L2, _L2)), cmap((_L2, _L2)),
        cmap((_L2, _L1)), cmap((_L2, _L1)),
        cmap((_L1, _N1)), cmap((_L1, _N1)),
    ]
    out = pl.pallas_call(
        _body,
        grid=(nd, b),
        in_specs=in_specs,
        out_specs=pl.BlockSpec((1, _L2, _DBLK, _N1),
                               lambda di, bi: (bi, 0, di, 0)),
        out_shape=jax.ShapeDtypeStruct((b, _L2, d, _N1), jnp.float32),
        scratch_shapes=[pltpu.VMEM((_L2, _DBLK, _L1), jnp.float32),
                        pltpu.VMEM((_L2, _DBLK, _L1), jnp.float32)],
        compiler_params=pltpu.CompilerParams(
            dimension_semantics=("parallel", "arbitrary"),
            vmem_limit_bytes=56 * 1024 * 1024,
        ),
    )(xv, tv, *consts)
    return out.transpose(0, 3, 1, 2).reshape(b, n, d)
